# baseline (speedup 1.0000x reference)
import functools

import jax
import jax.numpy as jnp
from jax import lax
from jax.experimental import pallas as pl
from jax.experimental.pallas import tpu as pltpu
from jax.experimental.pallas import tpu_sc as plsc

BATCH, SEQ, EMBED = 32, 2048, 32
_LANES = 16

_info = plsc.get_sparse_core_info()
_NC, _NS = _info.num_cores, _info.num_subcores
_NW = _NC * _NS

CHUNK = 1024
NCHUNK = SEQ // CHUNK
GSUB = 128
NG = CHUNK // GSUB


def _emb_body(x_hbm, tok_hbm, pos_hbm, out_hbm, idx_v, buf_v, pos_v,
              gsem, psem):
    w = lax.axis_index("s") * _NC + lax.axis_index("c")
    pltpu.sync_copy(x_hbm.at[w], idx_v)

    for c in range(NCHUNK):
        off = c * CHUNK
        pcp = pltpu.async_copy(pos_hbm.at[pl.ds(off, CHUNK)], pos_v, psem)
        cps = []
        for j in range(NG):
            cps.append(pltpu.async_copy(
                tok_hbm.at[idx_v.at[pl.ds(off + j * GSUB, GSUB)]],
                buf_v.at[pl.ds(j * GSUB, GSUB)],
                gsem))
        for cp in cps:
            cp.wait()
        pcp.wait()

        def add_row(i, carry):
            lo = pl.ds(0, _LANES)
            hi = pl.ds(_LANES, _LANES)
            plsc.addupdate(buf_v.at[i, lo], pos_v[i, lo])
            plsc.addupdate(buf_v.at[i, hi], pos_v[i, hi])
            return carry

        lax.fori_loop(0, CHUNK, add_row, 0)

        pltpu.sync_copy(buf_v, out_hbm.at[w, pl.ds(off, CHUNK)])


_mesh = plsc.VectorSubcoreMesh(core_axis_name="c", subcore_axis_name="s")

_emb = functools.partial(
    pl.kernel,
    mesh=_mesh,
    out_type=jax.ShapeDtypeStruct((BATCH, SEQ, EMBED), jnp.float32),
    compiler_params=pltpu.CompilerParams(use_tc_tiling_on_sc=False),
    scratch_types=[
        pltpu.VMEM((SEQ,), jnp.int32),
        pltpu.VMEM((CHUNK, EMBED), jnp.float32),
        pltpu.VMEM((CHUNK, EMBED), jnp.float32),
        pltpu.SemaphoreType.DMA,
        pltpu.SemaphoreType.DMA,
    ],
)(_emb_body)


def kernel(x, token_table, pos_table):
    return _emb(x.astype(jnp.int32), token_table, pos_table)
